# baseline (speedup 1.0000x reference)
import jax
import jax.numpy as jnp
from jax.experimental import pallas as pl
from jax.experimental.pallas import tpu as pltpu

B = 4096
X_DIM = 784
N_CLASSES = 10
HIDDEN = 2048
EMBED_DIM = 1024
LATENT_DIM = 64
K_CODES = 1024

TILE = 512
HALF = TILE // 2


def _fwd_kernel(x_ref, c_ref, we1_ref, be1_ref, we2_ref, be2_ref,
                wfc_ref, bfc_ref, cb_ref, wd1_ref, bd1_ref, wd2_ref,
                bd2_ref, recon_ref, ze_ref, zq_ref,
                s_e1, s_e2, s_d1, s_d2, s_cbhi, s_cblo, s_cbT):
    f32 = jnp.float32
    bf16 = jnp.bfloat16

    @pl.when(pl.program_id(1) == 0)
    def _prep():
        s_e1[...] = we1_ref[...].astype(bf16)
        s_e2[...] = we2_ref[...].astype(bf16)
        s_d1[...] = wd1_ref[...].astype(bf16)
        s_d2[...] = wd2_ref[...].astype(bf16)
        cb = cb_ref[...]
        cb_hi = cb.astype(bf16)
        s_cbhi[...] = cb_hi
        s_cblo[...] = (cb - cb_hi.astype(f32)).astype(bf16)
        s_cbT[...] = cb.T

    i = pl.program_id(0) * pl.num_programs(1) + pl.program_id(1)
    we1 = s_e1[...]
    we2 = s_e2[...]
    wd1 = s_d1[...]
    wd2 = s_d2[...]
    wfc = wfc_ref[...]
    cbT = s_cbT[...]
    cb2 = jnp.sum(cbT * cbT, axis=0, keepdims=True)
    cb_hi = s_cbhi[...]
    cb_lo = s_cblo[...]
    cls_iota = jax.lax.broadcasted_iota(jnp.int32, (HALF, N_CLASSES), 1)
    code_iota = jax.lax.broadcasted_iota(jnp.int32, (HALF, K_CODES), 1)

    def half(s):
        rows = pl.ds(s * HALF, HALF)
        x = x_ref[rows, :]
        cmat = c_ref[pl.ds(i * TILE + s * HALF, HALF)].reshape(HALF, 1)
        oh = (cmat == cls_iota).astype(bf16)

        xcat = jnp.concatenate([x, oh], axis=1)
        h = jnp.dot(xcat, we1, preferred_element_type=f32)
        h = jnp.maximum(h + be1_ref[...], 0.0)
        enc = jnp.maximum(
            jnp.dot(h.astype(bf16), we2, preferred_element_type=f32)
            + be2_ref[...], 0.0)
        z_e = jnp.dot(enc, wfc, preferred_element_type=f32) + bfc_ref[...]

        cross = jnp.dot(z_e, cbT, preferred_element_type=f32)
        d2 = cb2 - 2.0 * cross
        rowmin = jnp.min(d2, axis=-1, keepdims=True)
        cand = jnp.where(d2 == rowmin, code_iota, K_CODES)
        idx = jnp.min(cand, axis=-1, keepdims=True)
        qoh = (code_iota == idx).astype(bf16)
        quant = (jnp.dot(qoh, cb_hi, preferred_element_type=f32)
                 + jnp.dot(qoh, cb_lo, preferred_element_type=f32))
        z_q = z_e + (quant - z_e)

        zcat = jnp.concatenate([z_q.astype(bf16), oh], axis=1)
        hd = jnp.dot(zcat, wd1, preferred_element_type=f32)
        hd = jnp.maximum(hd + bd1_ref[...], 0.0)
        recon = jax.nn.sigmoid(
            jnp.dot(hd.astype(bf16), wd2, preferred_element_type=f32)
            + bd2_ref[...])

        recon_ref[rows, :] = recon
        ze_ref[rows, :] = z_e
        zq_ref[rows, :] = z_q

    half(0)
    half(1)


def kernel(x, c, W_e1, b_e1, W_e2, b_e2, W_fc, b_fc, codebook,
           W_d1, b_d1, W_d2, b_d2):
    n_tiles = B // TILE
    bf16 = jnp.bfloat16
    args = (
        x.astype(bf16),
        c.astype(jnp.int32),
        W_e1,
        b_e1.reshape(1, HIDDEN),
        W_e2,
        b_e2.reshape(1, EMBED_DIM),
        W_fc,
        b_fc.reshape(1, LATENT_DIM),
        codebook,
        W_d1,
        b_d1.reshape(1, HIDDEN),
        W_d2,
        b_d2.reshape(1, X_DIM),
    )

    n_inner = n_tiles // 2

    def tiled(ncols):
        return pl.BlockSpec((TILE, ncols), lambda i, j: (i * n_inner + j, 0))

    def whole(a):
        return pl.BlockSpec(a.shape, lambda i, j: tuple(0 for _ in a.shape))

    in_specs = [
        tiled(X_DIM),
    ] + [whole(a) for a in args[1:]]

    out_shape = (
        jax.ShapeDtypeStruct((B, X_DIM), jnp.float32),
        jax.ShapeDtypeStruct((B, LATENT_DIM), jnp.float32),
        jax.ShapeDtypeStruct((B, LATENT_DIM), jnp.float32),
    )
    out_specs = (
        tiled(X_DIM),
        tiled(LATENT_DIM),
        tiled(LATENT_DIM),
    )

    scratch_shapes = [
        pltpu.VMEM((X_DIM + N_CLASSES, HIDDEN), bf16),
        pltpu.VMEM((HIDDEN, EMBED_DIM), bf16),
        pltpu.VMEM((LATENT_DIM + N_CLASSES, HIDDEN), bf16),
        pltpu.VMEM((HIDDEN, X_DIM), bf16),
        pltpu.VMEM((K_CODES, LATENT_DIM), bf16),
        pltpu.VMEM((K_CODES, LATENT_DIM), bf16),
        pltpu.VMEM((LATENT_DIM, K_CODES), jnp.float32),
    ]

    recon, z_e, z_q = pl.pallas_call(
        _fwd_kernel,
        grid=(2, n_inner),
        in_specs=in_specs,
        out_specs=out_specs,
        out_shape=out_shape,
        scratch_shapes=scratch_shapes,
        compiler_params=pltpu.CompilerParams(
            dimension_semantics=("parallel", "arbitrary")),
    )(*args)
    return (recon, z_e, z_q)


# speedup vs baseline: 1.0535x; 1.0535x over previous
import jax
import jax.numpy as jnp
from jax.experimental import pallas as pl
from jax.experimental.pallas import tpu as pltpu

B = 4096
X_DIM = 784
N_CLASSES = 10
HIDDEN = 2048
EMBED_DIM = 1024
LATENT_DIM = 64
K_CODES = 1024

TILE = 512
HALF = TILE // 2


def _fwd_kernel(x_ref, c_ref, we1_ref, be1_ref, we2_ref, be2_ref,
                wfc_ref, bfc_ref, cb_ref, wd1_ref, bd1_ref, wd2_ref,
                bd2_ref, recon_ref, ze_ref, zq_ref,
                s_e1, s_e2, s_d1, s_d2, s_cbhi, s_cblo, s_cbT):
    f32 = jnp.float32
    bf16 = jnp.bfloat16

    @pl.when(pl.program_id(0) == 0)
    def _prep():
        s_e1[...] = we1_ref[...].astype(bf16)
        s_e2[...] = we2_ref[...].astype(bf16)
        s_d1[...] = wd1_ref[...].astype(bf16)
        s_d2[...] = wd2_ref[...].astype(bf16)
        cb = cb_ref[...]
        cb_hi = cb.astype(bf16)
        s_cbhi[...] = cb_hi
        s_cblo[...] = (cb - cb_hi.astype(f32)).astype(bf16)
        s_cbT[...] = cb.T

    i = pl.program_id(0)
    we1 = s_e1[...]
    we2 = s_e2[...]
    wd1 = s_d1[...]
    wd2 = s_d2[...]
    wfc = wfc_ref[...]
    cbT = s_cbT[...]
    cb2 = jnp.sum(cbT * cbT, axis=0, keepdims=True)
    cb_hi = s_cbhi[...]
    cb_lo = s_cblo[...]
    cls_iota = jax.lax.broadcasted_iota(jnp.int32, (HALF, N_CLASSES), 1)
    code_iota = jax.lax.broadcasted_iota(jnp.int32, (HALF, K_CODES), 1)

    def half(s):
        rows = pl.ds(s * HALF, HALF)
        x = x_ref[rows, :].astype(bf16)
        cmat = c_ref[pl.ds(i * TILE + s * HALF, HALF)].reshape(HALF, 1)
        oh = (cmat == cls_iota).astype(bf16)

        xcat = jnp.concatenate([x, oh], axis=1)
        h = jnp.dot(xcat, we1, preferred_element_type=f32)
        h = jnp.maximum(h + be1_ref[...].reshape(1, HIDDEN), 0.0)
        enc = jnp.maximum(
            jnp.dot(h.astype(bf16), we2, preferred_element_type=f32)
            + be2_ref[...].reshape(1, EMBED_DIM), 0.0)
        z_e = (jnp.dot(enc, wfc, preferred_element_type=f32)
               + bfc_ref[...].reshape(1, LATENT_DIM))

        cross = jnp.dot(z_e, cbT, preferred_element_type=f32)
        d2 = cb2 - 2.0 * cross
        rowmin = jnp.min(d2, axis=-1, keepdims=True)
        cand = jnp.where(d2 == rowmin, code_iota, K_CODES)
        idx = jnp.min(cand, axis=-1, keepdims=True)
        qoh = (code_iota == idx).astype(bf16)
        quant = (jnp.dot(qoh, cb_hi, preferred_element_type=f32)
                 + jnp.dot(qoh, cb_lo, preferred_element_type=f32))
        z_q = z_e + (quant - z_e)

        zcat = jnp.concatenate([z_q.astype(bf16), oh], axis=1)
        hd = jnp.dot(zcat, wd1, preferred_element_type=f32)
        hd = jnp.maximum(hd + bd1_ref[...].reshape(1, HIDDEN), 0.0)
        recon = jax.nn.sigmoid(
            jnp.dot(hd.astype(bf16), wd2, preferred_element_type=f32)
            + bd2_ref[...].reshape(1, X_DIM))

        recon_ref[rows, :] = recon
        ze_ref[rows, :] = z_e
        zq_ref[rows, :] = z_q

    half(0)
    half(1)


def kernel(x, c, W_e1, b_e1, W_e2, b_e2, W_fc, b_fc, codebook,
           W_d1, b_d1, W_d2, b_d2):
    n_tiles = B // TILE
    bf16 = jnp.bfloat16
    args = (
        x,
        c.astype(jnp.int32),
        W_e1,
        b_e1,
        W_e2,
        b_e2,
        W_fc,
        b_fc,
        codebook,
        W_d1,
        b_d1,
        W_d2,
        b_d2,
    )

    def tiled(ncols):
        return pl.BlockSpec((TILE, ncols), lambda i: (i, 0))

    def whole(a):
        return pl.BlockSpec(a.shape, lambda i: tuple(0 for _ in a.shape))

    in_specs = [
        tiled(X_DIM),
    ] + [whole(a) for a in args[1:]]

    out_shape = (
        jax.ShapeDtypeStruct((B, X_DIM), jnp.float32),
        jax.ShapeDtypeStruct((B, LATENT_DIM), jnp.float32),
        jax.ShapeDtypeStruct((B, LATENT_DIM), jnp.float32),
    )
    out_specs = (
        tiled(X_DIM),
        tiled(LATENT_DIM),
        tiled(LATENT_DIM),
    )

    scratch_shapes = [
        pltpu.VMEM((X_DIM + N_CLASSES, HIDDEN), bf16),
        pltpu.VMEM((HIDDEN, EMBED_DIM), bf16),
        pltpu.VMEM((LATENT_DIM + N_CLASSES, HIDDEN), bf16),
        pltpu.VMEM((HIDDEN, X_DIM), bf16),
        pltpu.VMEM((K_CODES, LATENT_DIM), bf16),
        pltpu.VMEM((K_CODES, LATENT_DIM), bf16),
        pltpu.VMEM((LATENT_DIM, K_CODES), jnp.float32),
    ]

    recon, z_e, z_q = pl.pallas_call(
        _fwd_kernel,
        grid=(n_tiles,),
        in_specs=in_specs,
        out_specs=out_specs,
        out_shape=out_shape,
        scratch_shapes=scratch_shapes,
    )(*args)
    return (recon, z_e, z_q)
